# Initial kernel scaffold; baseline (speedup 1.0000x reference)
import jax
import jax.numpy as jnp
from jax.experimental import pallas as pl


def kernel(x, W_conv, b_conv):
    raise NotImplementedError("write your pallas kernel here")


# speedup vs baseline: 7.3997x; 7.3997x over previous
import jax
import jax.numpy as jnp
from jax.experimental import pallas as pl

B, C, N, P = 8, 192, 576, 768
_BIG = 3.0e38


def _tc_body(xp_ref, w_ref, b_ref, y_ref, adj_ref):
    xb = xp_ref[0]
    w = w_ref[...]
    bias = b_ref[...]
    yb = jnp.dot(xb, w, preferred_element_type=jnp.float32) + bias
    y_ref[0] = yb
    g = jax.lax.dot_general(yb, yb, (((1,), (1,)), ((), ())),
                            preferred_element_type=jnp.float32)
    sq = jnp.sum(yb * yb, axis=1)
    work = jnp.maximum(sq[:, None] + sq[None, :] - 2.0 * g, 0.0)
    col = jax.lax.broadcasted_iota(jnp.int32, (N, N), 1)
    adj = jnp.zeros((N, N), jnp.float32)
    for _ in range(7):
        m = jnp.min(work, axis=1, keepdims=True)
        cand = jnp.where(work == m, col, jnp.int32(1 << 30))
        idx = jnp.min(cand, axis=1, keepdims=True)
        sel = cand == idx
        adj = jnp.where(sel, jnp.float32(1.0), adj)
        work = jnp.where(sel, _BIG, work)
    adj_ref[0] = adj


def kernel(x, W_conv, b_conv):
    xp = (x.reshape(B, 3, 24, 16, 24, 16)
            .transpose(0, 2, 4, 1, 3, 5)
            .reshape(B, N, P))
    wm = W_conv.reshape(C, P).T
    bias = b_conv.reshape(1, C)
    y, adj = pl.pallas_call(
        _tc_body,
        grid=(B,),
        in_specs=[
            pl.BlockSpec((1, N, P), lambda b: (b, 0, 0)),
            pl.BlockSpec((P, C), lambda b: (0, 0)),
            pl.BlockSpec((1, C), lambda b: (0, 0)),
        ],
        out_specs=[
            pl.BlockSpec((1, N, C), lambda b: (b, 0, 0)),
            pl.BlockSpec((1, N, N), lambda b: (b, 0, 0)),
        ],
        out_shape=[
            jax.ShapeDtypeStruct((B, N, C), jnp.float32),
            jax.ShapeDtypeStruct((B, N, N), jnp.float32),
        ],
    )(xp, wm, bias)
    return (y, adj)
